# Initial kernel scaffold; baseline (speedup 1.0000x reference)
import jax
import jax.numpy as jnp
from jax.experimental import pallas as pl


def kernel(pos, batch, sa1_W1, sa1_b1, sa1_W2, sa1_b2, sa2_W1, sa2_b1, sa2_W2, sa2_b2):
    raise NotImplementedError("write your pallas kernel here")


# speedup vs baseline: 2.6626x; 2.6626x over previous
import jax
import jax.numpy as jnp
from jax.experimental import pallas as pl

B, P = 4, 256
C1, C2 = P // 2, P // 8
R1, R2 = 0.2, 0.4
NEG = -1e9


def _fps_level(px, py, pz, n, cidx_n):
    bsz, npts = px.shape
    jidx = jax.lax.broadcasted_iota(jnp.int32, (bsz, npts), 1)
    qx0 = px[:, 0:1]
    qy0 = py[:, 0:1]
    qz0 = pz[:, 0:1]
    dx = px - qx0
    dy = py - qy0
    dz = pz - qz0
    dist = dx * dx + dy * dy + dz * dz
    col0 = cidx_n == 0
    self_f = jnp.zeros((bsz, n), jnp.float32)
    cx = jnp.where(col0, qx0, 0.0)
    cy = jnp.where(col0, qy0, 0.0)
    cz = jnp.where(col0, qz0, 0.0)

    def body(i, state):
        self_f, cx, cy, cz, dist = state
        m = jnp.max(dist, axis=1, keepdims=True)
        eq = dist == m
        nxt = jnp.min(jnp.where(eq, jidx, npts), axis=1, keepdims=True)
        pick = jidx == nxt
        qx = jnp.max(jnp.where(pick, px, -jnp.inf), axis=1, keepdims=True)
        qy = jnp.max(jnp.where(pick, py, -jnp.inf), axis=1, keepdims=True)
        qz = jnp.max(jnp.where(pick, pz, -jnp.inf), axis=1, keepdims=True)
        dx = px - qx
        dy = py - qy
        dz = pz - qz
        d2 = dx * dx + dy * dy + dz * dz
        dist = jnp.minimum(dist, d2)
        col = cidx_n == i
        nxtf = nxt.astype(jnp.float32)
        self_f = jnp.where(col, nxtf, self_f)
        cx = jnp.where(col, qx, cx)
        cy = jnp.where(col, qy, cy)
        cz = jnp.where(col, qz, cz)
        return (self_f, cx, cy, cz, dist)

    self_f, cx, cy, cz, _ = jax.lax.fori_loop(
        1, n, body, (self_f, cx, cy, cz, dist))
    return self_f, cx, cy, cz


def _fps_body(posT_ref, sel1_ref, cen1T_ref, sel2_ref, cen2T_ref):
    px = posT_ref[0]
    py = posT_ref[1]
    pz = posT_ref[2]
    cidx1 = jax.lax.broadcasted_iota(jnp.int32, (B, C1), 1)
    s1, cx1, cy1, cz1 = _fps_level(px, py, pz, C1, cidx1)
    sel1_ref[...] = s1
    cen1T_ref[0] = cx1
    cen1T_ref[1] = cy1
    cen1T_ref[2] = cz1
    cidx2 = jax.lax.broadcasted_iota(jnp.int32, (B, C2), 1)
    s2, cx2, cy2, cz2 = _fps_level(cx1, cy1, cz1, C2, cidx2)
    sel2_ref[...] = s2
    cen2T_ref[0] = cx2
    cen2T_ref[1] = cy2
    cen2T_ref[2] = cz2


def _stage1_body(pos_ref, cen_ref, w1_ref, b1_ref, w2_ref, b2_ref, x1_ref):
    pos2 = pos_ref[0]
    cen4 = cen_ref[0]
    cen3 = cen4[:, 0:3]
    w1 = w1_ref[...]
    a = jnp.dot(pos2, w1, preferred_element_type=jnp.float32)
    d = jnp.dot(cen3, w1, preferred_element_type=jnp.float32)
    pre = a[None, :, :] - d[:, None, :] + b1_ref[...]
    h = jax.nn.relu(pre).reshape(C1 * P, 64)
    msg = jnp.dot(h, w2_ref[...], preferred_element_type=jnp.float32)
    msg = msg.reshape(C1, P, 128) + b2_ref[...]

    dx = pos2[:, 0:1][None] - cen4[:, 0:1][:, None]
    dy = pos2[:, 1:2][None] - cen4[:, 1:2][:, None]
    dz = pos2[:, 2:3][None] - cen4[:, 2:3][:, None]
    sq = dx * dx + dy * dy + dz * dz
    jidx = jax.lax.broadcasted_iota(jnp.int32, (C1, P, 1), 1)
    mask = (sq < R1 * R1) & (jidx != cen4[:, 3:4].astype(jnp.int32)[:, None])
    m1 = jnp.max(jnp.where(mask, msg, NEG), axis=1)
    x1_ref[0] = jnp.where(jnp.any(mask, axis=1), m1, 0.0)


def _stage2_body(x1_ref, cen1_ref, cen2_ref, w1a_ref, w1b_ref,
                 b1_ref, w2_ref, b2_ref, out_ref):
    x1 = x1_ref[0]
    cen1 = cen1_ref[0]
    cen2 = cen2_ref[0]
    w1b = w1b_ref[...]
    u = (jnp.dot(x1, w1a_ref[...], preferred_element_type=jnp.float32)
         + jnp.dot(cen1[:, 0:3], w1b, preferred_element_type=jnp.float32)
         + b1_ref[...])
    v = jnp.dot(cen2[:, 0:3], w1b, preferred_element_type=jnp.float32)
    pre = u[None, :, :] - v[:, None, :]
    h = jax.nn.relu(pre).reshape(C2 * C1, 256)
    msg = jnp.dot(h, w2_ref[...], preferred_element_type=jnp.float32)
    msg = msg.reshape(C2, C1, 512) + b2_ref[...]

    dx = cen1[:, 0:1][None] - cen2[:, 0:1][:, None]
    dy = cen1[:, 1:2][None] - cen2[:, 1:2][:, None]
    dz = cen1[:, 2:3][None] - cen2[:, 2:3][:, None]
    sq = dx * dx + dy * dy + dz * dz
    jidx = jax.lax.broadcasted_iota(jnp.int32, (C2, C1, 1), 1)
    mask = (sq < R2 * R2) & (jidx != cen2[:, 3:4].astype(jnp.int32)[:, None])
    m2 = jnp.max(jnp.where(mask, msg, NEG), axis=1)
    x2 = jnp.where(jnp.any(mask, axis=1), m2, 0.0)
    out_ref[0, 0] = jnp.max(x2, axis=0)


def kernel(pos, batch, sa1_W1, sa1_b1, sa1_W2, sa1_b2, sa2_W1, sa2_b1,
           sa2_W2, sa2_b2):
    del batch
    pos_b = pos.reshape(B, P, 3)
    posT = jnp.transpose(pos_b, (2, 0, 1))

    sel1, cen1T, sel2, cen2T = pl.pallas_call(
        _fps_body,
        out_shape=[
            jax.ShapeDtypeStruct((B, C1), jnp.float32),
            jax.ShapeDtypeStruct((3, B, C1), jnp.float32),
            jax.ShapeDtypeStruct((B, C2), jnp.float32),
            jax.ShapeDtypeStruct((3, B, C2), jnp.float32),
        ],
    )(posT)

    cen1aug = jnp.concatenate(
        [jnp.transpose(cen1T, (1, 2, 0)), sel1[:, :, None]], axis=-1)
    cen2aug = jnp.concatenate(
        [jnp.transpose(cen2T, (1, 2, 0)), sel2[:, :, None]], axis=-1)

    b1r = sa1_b1.reshape(1, 64)
    b2r = sa1_b2.reshape(1, 128)
    x1 = pl.pallas_call(
        _stage1_body,
        grid=(B,),
        in_specs=[
            pl.BlockSpec((1, P, 3), lambda b: (b, 0, 0)),
            pl.BlockSpec((1, C1, 4), lambda b: (b, 0, 0)),
            pl.BlockSpec((3, 64), lambda b: (0, 0)),
            pl.BlockSpec((1, 64), lambda b: (0, 0)),
            pl.BlockSpec((64, 128), lambda b: (0, 0)),
            pl.BlockSpec((1, 128), lambda b: (0, 0)),
        ],
        out_specs=pl.BlockSpec((1, C1, 128), lambda b: (b, 0, 0)),
        out_shape=jax.ShapeDtypeStruct((B, C1, 128), jnp.float32),
    )(pos_b, cen1aug, sa1_W1, b1r, sa1_W2, b2r)

    w1a = sa2_W1[:128]
    w1b = sa2_W1[128:]
    b1r2 = sa2_b1.reshape(1, 256)
    b2r2 = sa2_b2.reshape(1, 512)
    out = pl.pallas_call(
        _stage2_body,
        grid=(B,),
        in_specs=[
            pl.BlockSpec((1, C1, 128), lambda b: (b, 0, 0)),
            pl.BlockSpec((1, C1, 4), lambda b: (b, 0, 0)),
            pl.BlockSpec((1, C2, 4), lambda b: (b, 0, 0)),
            pl.BlockSpec((128, 256), lambda b: (0, 0)),
            pl.BlockSpec((3, 256), lambda b: (0, 0)),
            pl.BlockSpec((1, 256), lambda b: (0, 0)),
            pl.BlockSpec((256, 512), lambda b: (0, 0)),
            pl.BlockSpec((1, 512), lambda b: (0, 0)),
        ],
        out_specs=pl.BlockSpec((1, 1, 512), lambda b: (b, 0, 0)),
        out_shape=jax.ShapeDtypeStruct((B, 1, 512), jnp.float32),
    )(x1, cen1aug, cen2aug, w1a, w1b, b1r2, sa2_W2, b2r2)

    return out.reshape(B, 512)


# speedup vs baseline: 2.8813x; 1.0821x over previous
import jax
import jax.numpy as jnp
from jax.experimental import pallas as pl

B, P = 4, 256
C1, C2 = P // 2, P // 8
R1, R2 = 0.2, 0.4
NEG = -1e9
THRESH = -5e8


def _fps_level(px, py, pz, n, cidx_n):
    bsz, npts = px.shape
    jidx = jax.lax.broadcasted_iota(jnp.int32, (bsz, npts), 1)
    qx0 = px[:, 0:1]
    qy0 = py[:, 0:1]
    qz0 = pz[:, 0:1]
    dx = px - qx0
    dy = py - qy0
    dz = pz - qz0
    dist = dx * dx + dy * dy + dz * dz
    col0 = cidx_n == 0
    self_f = jnp.zeros((bsz, n), jnp.float32)
    cx = jnp.where(col0, qx0, 0.0)
    cy = jnp.where(col0, qy0, 0.0)
    cz = jnp.where(col0, qz0, 0.0)

    def body(i, state):
        self_f, cx, cy, cz, dist = state
        m = jnp.max(dist, axis=1, keepdims=True)
        eq = dist == m
        nxt = jnp.min(jnp.where(eq, jidx, npts), axis=1, keepdims=True)
        pick = jidx == nxt
        qx = jnp.max(jnp.where(pick, px, -jnp.inf), axis=1, keepdims=True)
        qy = jnp.max(jnp.where(pick, py, -jnp.inf), axis=1, keepdims=True)
        qz = jnp.max(jnp.where(pick, pz, -jnp.inf), axis=1, keepdims=True)
        dx = px - qx
        dy = py - qy
        dz = pz - qz
        d2 = dx * dx + dy * dy + dz * dz
        dist = jnp.minimum(dist, d2)
        col = cidx_n == i
        nxtf = nxt.astype(jnp.float32)
        self_f = jnp.where(col, nxtf, self_f)
        cx = jnp.where(col, qx, cx)
        cy = jnp.where(col, qy, cy)
        cz = jnp.where(col, qz, cz)
        return (self_f, cx, cy, cz, dist)

    self_f, cx, cy, cz, _ = jax.lax.fori_loop(
        1, n, body, (self_f, cx, cy, cz, dist))
    return self_f, cx, cy, cz


def _fps_body(posT_ref, sel1_ref, cen1T_ref, sel2_ref, cen2T_ref):
    px = posT_ref[0]
    py = posT_ref[1]
    pz = posT_ref[2]
    cidx1 = jax.lax.broadcasted_iota(jnp.int32, (B, C1), 1)
    s1, cx1, cy1, cz1 = _fps_level(px, py, pz, C1, cidx1)
    sel1_ref[...] = s1
    cen1T_ref[0] = cx1
    cen1T_ref[1] = cy1
    cen1T_ref[2] = cz1
    cidx2 = jax.lax.broadcasted_iota(jnp.int32, (B, C2), 1)
    s2, cx2, cy2, cz2 = _fps_level(cx1, cy1, cz1, C2, cidx2)
    sel2_ref[...] = s2
    cen2T_ref[0] = cx2
    cen2T_ref[1] = cy2
    cen2T_ref[2] = cz2


def _stages_body(pos_ref, cen1_ref, cen2_ref, w1_ref, b1_ref, w2a_ref,
                 b2_ref, w21a_ref, w21b_ref, b21_ref, w22a_ref, b22_ref,
                 out_ref):
    pos2 = pos_ref[0]
    cen4 = cen1_ref[0]
    cen3 = cen4[:, 0:3]
    w1 = w1_ref[...]
    a = jnp.dot(pos2, w1, preferred_element_type=jnp.float32)
    d = jnp.dot(cen3, w1, preferred_element_type=jnp.float32)
    pre = a[None, :, :] - d[:, None, :] + b1_ref[...]
    h = jax.nn.relu(pre)

    dx = pos2[:, 0:1][None] - cen4[:, 0:1][:, None]
    dy = pos2[:, 1:2][None] - cen4[:, 1:2][:, None]
    dz = pos2[:, 2:3][None] - cen4[:, 2:3][:, None]
    sq = dx * dx + dy * dy + dz * dz
    jidx = jax.lax.broadcasted_iota(jnp.int32, (C1, P, 1), 1)
    mask = (sq < R1 * R1) & (jidx != cen4[:, 3:4].astype(jnp.int32)[:, None])
    pen = jnp.where(mask, 0.0, NEG)
    h_aug = jnp.concatenate([h, pen], axis=2).reshape(C1 * P, 65)
    msg = jnp.dot(h_aug, w2a_ref[...], preferred_element_type=jnp.float32)
    m1 = jnp.max(msg.reshape(C1, P, 128), axis=1)
    x1 = jnp.where(m1 > THRESH, m1 + b2_ref[...], 0.0)

    cen24 = cen2_ref[0]
    w21b = w21b_ref[...]
    u = (jnp.dot(x1, w21a_ref[...], preferred_element_type=jnp.float32)
         + jnp.dot(cen3, w21b, preferred_element_type=jnp.float32)
         + b21_ref[...])
    v = jnp.dot(cen24[:, 0:3], w21b, preferred_element_type=jnp.float32)
    pre2 = u[None, :, :] - v[:, None, :]
    h2 = jax.nn.relu(pre2)

    dx2 = cen4[:, 0:1][None] - cen24[:, 0:1][:, None]
    dy2 = cen4[:, 1:2][None] - cen24[:, 1:2][:, None]
    dz2 = cen4[:, 2:3][None] - cen24[:, 2:3][:, None]
    sq2 = dx2 * dx2 + dy2 * dy2 + dz2 * dz2
    jidx2 = jax.lax.broadcasted_iota(jnp.int32, (C2, C1, 1), 1)
    mask2 = ((sq2 < R2 * R2)
             & (jidx2 != cen24[:, 3:4].astype(jnp.int32)[:, None]))
    pen2 = jnp.where(mask2, 0.0, NEG)
    h2_aug = jnp.concatenate([h2, pen2], axis=2).reshape(C2 * C1, 257)
    msg2 = jnp.dot(h2_aug, w22a_ref[...], preferred_element_type=jnp.float32)
    m2 = jnp.max(msg2.reshape(C2, C1, 512), axis=1)
    x2 = jnp.where(m2 > THRESH, m2 + b22_ref[...], 0.0)
    out_ref[0, 0] = jnp.max(x2, axis=0)


def kernel(pos, batch, sa1_W1, sa1_b1, sa1_W2, sa1_b2, sa2_W1, sa2_b1,
           sa2_W2, sa2_b2):
    del batch
    pos_b = pos.reshape(B, P, 3)
    posT = jnp.transpose(pos_b, (2, 0, 1))

    sel1, cen1T, sel2, cen2T = pl.pallas_call(
        _fps_body,
        out_shape=[
            jax.ShapeDtypeStruct((B, C1), jnp.float32),
            jax.ShapeDtypeStruct((3, B, C1), jnp.float32),
            jax.ShapeDtypeStruct((B, C2), jnp.float32),
            jax.ShapeDtypeStruct((3, B, C2), jnp.float32),
        ],
    )(posT)

    cen1aug = jnp.concatenate(
        [jnp.transpose(cen1T, (1, 2, 0)), sel1[:, :, None]], axis=-1)
    cen2aug = jnp.concatenate(
        [jnp.transpose(cen2T, (1, 2, 0)), sel2[:, :, None]], axis=-1)

    ones1 = jnp.ones((1, 128), jnp.float32)
    ones2 = jnp.ones((1, 512), jnp.float32)
    w2aug = jnp.concatenate([sa1_W2, ones1], axis=0)
    w22aug = jnp.concatenate([sa2_W2, ones2], axis=0)

    out = pl.pallas_call(
        _stages_body,
        grid=(B,),
        in_specs=[
            pl.BlockSpec((1, P, 3), lambda b: (b, 0, 0)),
            pl.BlockSpec((1, C1, 4), lambda b: (b, 0, 0)),
            pl.BlockSpec((1, C2, 4), lambda b: (b, 0, 0)),
            pl.BlockSpec((3, 64), lambda b: (0, 0)),
            pl.BlockSpec((1, 64), lambda b: (0, 0)),
            pl.BlockSpec((65, 128), lambda b: (0, 0)),
            pl.BlockSpec((1, 128), lambda b: (0, 0)),
            pl.BlockSpec((128, 256), lambda b: (0, 0)),
            pl.BlockSpec((3, 256), lambda b: (0, 0)),
            pl.BlockSpec((1, 256), lambda b: (0, 0)),
            pl.BlockSpec((257, 512), lambda b: (0, 0)),
            pl.BlockSpec((1, 512), lambda b: (0, 0)),
        ],
        out_specs=pl.BlockSpec((1, 1, 512), lambda b: (b, 0, 0)),
        out_shape=jax.ShapeDtypeStruct((B, 1, 512), jnp.float32),
    )(pos_b, cen1aug, cen2aug, sa1_W1, sa1_b1.reshape(1, 64), w2aug,
      sa1_b2.reshape(1, 128), sa2_W1[:128], sa2_W1[128:],
      sa2_b1.reshape(1, 256), w22aug, sa2_b2.reshape(1, 512))

    return out.reshape(B, 512)
